# Initial kernel scaffold; baseline (speedup 1.0000x reference)
import jax
import jax.numpy as jnp
from jax.experimental import pallas as pl


def kernel(x, pos_table):
    raise NotImplementedError("write your pallas kernel here")


# speedup vs baseline: 1.8754x; 1.8754x over previous
import jax
import jax.numpy as jnp
from jax.experimental import pallas as pl

_BS = 1024


def _add_kernel(x_ref, pos_ref, o_ref):
    o_ref[...] = x_ref[...] + pos_ref[...][None, :, :]


def kernel(x, pos_table):
    B, L, D = x.shape
    grid = (L // _BS, B)
    return pl.pallas_call(
        _add_kernel,
        grid=grid,
        in_specs=[
            pl.BlockSpec((1, _BS, D), lambda i, b: (b, i, 0)),
            pl.BlockSpec((_BS, D), lambda i, b: (i, 0)),
        ],
        out_specs=pl.BlockSpec((1, _BS, D), lambda i, b: (b, i, 0)),
        out_shape=jax.ShapeDtypeStruct((B, L, D), x.dtype),
    )(x, pos_table)


# speedup vs baseline: 1.9900x; 1.0611x over previous
import jax
import jax.numpy as jnp
from jax.experimental import pallas as pl

_BS = 2048


def _add_kernel(x_ref, pos_ref, o_ref):
    o_ref[...] = x_ref[...] + pos_ref[...][None, :, :]


def kernel(x, pos_table):
    B, L, D = x.shape
    grid = (L // _BS, B)
    return pl.pallas_call(
        _add_kernel,
        grid=grid,
        in_specs=[
            pl.BlockSpec((1, _BS, D), lambda i, b: (b, i, 0)),
            pl.BlockSpec((_BS, D), lambda i, b: (i, 0)),
        ],
        out_specs=pl.BlockSpec((1, _BS, D), lambda i, b: (b, i, 0)),
        out_shape=jax.ShapeDtypeStruct((B, L, D), x.dtype),
    )(x, pos_table)


# speedup vs baseline: 2.0001x; 1.0051x over previous
import jax
import jax.numpy as jnp
from jax.experimental import pallas as pl
from jax.experimental.pallas import tpu as pltpu

_BS = 2048


def _add_kernel(x_ref, pos_ref, o_ref):
    o_ref[...] = x_ref[...] + pos_ref[...][None, :, :]


def kernel(x, pos_table):
    B, L, D = x.shape
    grid = (L // _BS, B)
    return pl.pallas_call(
        _add_kernel,
        grid=grid,
        in_specs=[
            pl.BlockSpec((1, _BS, D), lambda i, b: (b, i, 0)),
            pl.BlockSpec((_BS, D), lambda i, b: (i, 0)),
        ],
        out_specs=pl.BlockSpec((1, _BS, D), lambda i, b: (b, i, 0)),
        out_shape=jax.ShapeDtypeStruct((B, L, D), x.dtype),
        compiler_params=pltpu.CompilerParams(vmem_limit_bytes=120 * 1024 * 1024),
    )(x, pos_table)
